# baseline (speedup 1.0000x reference)
import functools

import jax
import jax.numpy as jnp
from jax import lax
from jax.experimental import pallas as pl
from jax.experimental.pallas import tpu as pltpu
from jax.experimental.pallas import tpu_sc as plsc

EMBED = 256
CODES = 400

N_TOKENS = 32 * 576
NC, NS = 2, 16
NW = NC * NS
BPW = N_TOKENS // NW
CH = 96
NCH = BPW // CH


def _argmin_block(x_ref, e_ref, esq_ref, out_ref):
    x = x_ref[...]
    e = e_ref[...]
    d = esq_ref[...] - 2.0 * jax.lax.dot_general(
        x, e, (((1,), (1,)), ((), ())), preferred_element_type=jnp.float32)
    m = jnp.min(d, axis=1, keepdims=True)
    col = lax.broadcasted_iota(jnp.int32, d.shape, 1)
    idx = jnp.min(jnp.where(d <= m, col, CODES), axis=1)
    out_ref[...] = idx[:, None]


def _tc_indices(flat, embeddings):
    e_sq = jnp.sum(embeddings * embeddings, axis=1)[None, :]
    n = flat.shape[0]
    blk = 1152
    idx = pl.pallas_call(
        _argmin_block,
        grid=(n // blk,),
        in_specs=[
            pl.BlockSpec((blk, EMBED), lambda i: (i, 0)),
            pl.BlockSpec((CODES, EMBED), lambda i: (0, 0)),
            pl.BlockSpec((1, CODES), lambda i: (0, 0)),
        ],
        out_specs=pl.BlockSpec((blk, 1), lambda i: (i, 0)),
        out_shape=jax.ShapeDtypeStruct((n, 1), jnp.int32),
    )(flat, embeddings, e_sq)
    return idx.reshape(n)


def _sc_gather_body(table_hbm, idx_hbm, out_hbm, idx_v, buf0, buf1, sem0, sem1):
    wid = lax.axis_index("s") * NC + lax.axis_index("c")
    base = wid * BPW
    pltpu.sync_copy(idx_hbm.at[wid], idx_v)
    bufs = (buf0, buf1)
    sems = (sem0, sem1)
    copies = [None, None]
    copies[0] = pltpu.async_copy(table_hbm.at[idx_v.at[0]], bufs[0], sems[0])
    for c in range(NCH):
        nxt = c + 1
        if nxt < NCH:
            copies[nxt % 2] = pltpu.async_copy(
                table_hbm.at[idx_v.at[nxt]], bufs[nxt % 2], sems[nxt % 2])
        copies[c % 2].wait()
        pltpu.sync_copy(bufs[c % 2], out_hbm.at[pl.ds(base + c * CH, CH)])


def _sc_gather():
    return pl.kernel(
        _sc_gather_body,
        out_type=jax.ShapeDtypeStruct((N_TOKENS, EMBED), jnp.float32),
        mesh=plsc.VectorSubcoreMesh(core_axis_name="c", subcore_axis_name="s"),
        scratch_types=[
            pltpu.VMEM((NCH, CH), jnp.int32),
            pltpu.VMEM((CH, EMBED), jnp.float32),
            pltpu.VMEM((CH, EMBED), jnp.float32),
            pltpu.SemaphoreType.DMA,
            pltpu.SemaphoreType.DMA,
        ],
    )


def kernel(x, embeddings):
    flat = x.reshape(-1, EMBED)
    idx = _tc_indices(flat, embeddings)
    q = _sc_gather()(embeddings, idx.reshape(NW, NCH, CH))
    return q.reshape(x.shape)


# speedup vs baseline: 2.9039x; 2.9039x over previous
import jax
import jax.numpy as jnp
from jax import lax
from jax.experimental import pallas as pl

EMBED = 256
CODES = 400


def _vq_block(x_ref, e_ref, esq_ref, out_ref):
    x = x_ref[...]
    e = e_ref[...]
    d = esq_ref[...] - 2.0 * jax.lax.dot_general(
        x, e, (((1,), (1,)), ((), ())), preferred_element_type=jnp.float32)
    m = jnp.min(d, axis=1, keepdims=True)
    col = lax.broadcasted_iota(jnp.int32, d.shape, 1)
    idx = jnp.min(jnp.where(d <= m, col, CODES), axis=1)
    onehot = (col == idx[:, None]).astype(jnp.float32)
    out_ref[...] = jax.lax.dot_general(
        onehot, e, (((1,), (0,)), ((), ())), preferred_element_type=jnp.float32)


def kernel(x, embeddings):
    flat = x.reshape(-1, EMBED)
    n = flat.shape[0]
    e_sq = jnp.sum(embeddings * embeddings, axis=1)[None, :]
    blk = 2304
    q = pl.pallas_call(
        _vq_block,
        grid=(n // blk,),
        in_specs=[
            pl.BlockSpec((blk, EMBED), lambda i: (i, 0)),
            pl.BlockSpec((CODES, EMBED), lambda i: (0, 0)),
            pl.BlockSpec((1, CODES), lambda i: (0, 0)),
        ],
        out_specs=pl.BlockSpec((blk, EMBED), lambda i: (i, 0)),
        out_shape=jax.ShapeDtypeStruct((n, EMBED), jnp.float32),
    )(flat, embeddings, e_sq)
    return q.reshape(x.shape)
